# baseline (speedup 1.0000x reference)
import jax
import jax.numpy as jnp
from jax import lax
from jax.experimental import pallas as pl
from jax.experimental.pallas import tpu as pltpu
from jax.experimental.pallas import tpu_sc as plsc

N = 10000
E = 320000
D = 128
AVG_D = 3.5

NC = 2
NS = 16
NW = NC * NS
NPASS = 2
RP = 160
SPAN = NW * RP
N_PAD = NPASS * SPAN
C = 1280
NCHUNK = E // C
K = 64
LCAP = C + K
NEG = -3.4e38
POS = 3.4e38

N_OUT = 16512
N_CNT = 131200
E_PAD = 2200000

EPI_BLK = 1000


def _roll_gather(v, idx):
    return lax.gather(
        v, idx[:, None],
        dimension_numbers=lax.GatherDimensionNumbers(
            offset_dims=(), collapsed_slice_dims=(0,), start_index_map=(0,)),
        slice_sizes=(1,),
        mode=lax.GatherScatterMode.PROMISE_IN_BOUNDS)


def _sc_body(edges_h, u_h, w_h, sum_o, mx_o, mn_o, cnt_o,
             dbuf, sbuf, posb, dlocb, eidb, gd_l, src_l, eid_l,
             gd_f, src_f, eid_f, urows, wrows,
             sumacc, mxacc, mnacc, cntacc, exb, sem_g):
    c_ax = lax.axis_index("c")
    s_ax = lax.axis_index("s")
    wid = c_ax * NS + s_ax
    s_off = s_ax * LCAP
    iota = lax.iota(jnp.int32, 16)
    ones = jnp.full((16,), 1, jnp.int32)
    zl = jnp.full((16,), 0, jnp.int32)
    f32 = jnp.float32

    for p in range(NPASS):
        lo = p * SPAN + wid * RP
        hi = lo + RP

        def init_acc(r, _):
            for j in range(D // 16):
                sl = pl.ds(j * 16, 16)
                sumacc[r, sl] = jnp.full((16,), 0.0, f32)
                mxacc[r, sl] = jnp.full((16,), NEG, f32)
                mnacc[r, sl] = jnp.full((16,), POS, f32)
            cntacc[r, :] = jnp.full((16,), 0.0, f32)
            return 0
        lax.fori_loop(0, RP + 8, init_acc, 0)

        def process_batch(i, is_tail, rem_vec=None):
            off = s_off + i * K
            pltpu.sync_copy(gd_l.at[pl.ds(off, K)], gd_f)
            pltpu.sync_copy(src_l.at[pl.ds(off, K)], src_f)
            pltpu.sync_copy(eid_l.at[pl.ds(off, K)], eid_f)
            if is_tail:
                for j in range(K // 16):
                    sl = pl.ds(j * 16, 16)
                    lid = jnp.full((16,), j * 16, jnp.int32) + iota
                    valid = lid < rem_vec
                    gd_f[sl] = jnp.where(valid, gd_f[sl],
                                         jnp.full((16,), RP, jnp.int32))
                    src_f[sl] = jnp.where(valid, src_f[sl], zl)
                    eid_f[sl] = jnp.where(valid, eid_f[sl], zl)
            cp_u = pltpu.async_copy(u_h.at[src_f], urows, sem_g)
            cp_w = pltpu.async_copy(w_h.at[eid_f], wrows, sem_g)
            cp_u.wait()
            cp_w.wait()

            def q_body(q, _):
                gdv = gd_f[pl.ds(q * 16, 16)]
                ldv = jnp.minimum(jnp.maximum(gdv, 0), RP)
                exb[...] = ldv
                lvec = exb[...]
                for lane in range(16):
                    ld = lvec[lane]
                    row = q * 16 + lane
                    cntacc[ld, :] = cntacc[ld, :] + jnp.full((16,), 1.0, f32)
                    for j in range(D // 16):
                        sl = pl.ds(j * 16, 16)
                        gv = urows[row, sl] + wrows[row, sl]
                        sumacc[ld, sl] = sumacc[ld, sl] + gv
                        mxacc[ld, sl] = jnp.maximum(mxacc[ld, sl], gv)
                        mnacc[ld, sl] = jnp.minimum(mnacc[ld, sl], gv)
                return 0
            lax.fori_loop(0, K // 16, q_body, 0)

        def chunk_body(c, bvec):
            pltpu.sync_copy(edges_h.at[pl.ds(c * C, C)], dbuf)
            pltpu.sync_copy(edges_h.at[pl.ds(E + c * C, C)], sbuf)

            def sg_body(sg, bv):
                def g8_body(g8, bv2):
                    b16 = sg * 128 + g8 * 16
                    d = dbuf[pl.ds(b16, 16)]
                    m = (d >= lo) & (d < hi)
                    mi = jnp.where(m, ones, zl)
                    t = mi
                    for k in (1, 2, 4, 8):
                        t = t + _roll_gather(t, lax.rem(iota + k, 16))
                    pr = mi
                    for k in (1, 2, 4, 8):
                        sh = _roll_gather(pr, jnp.maximum(iota - k, 0))
                        pr = pr + jnp.where(iota >= k, sh, zl)
                    pos = jnp.where(m, bv2 + pr - ones + s_off,
                                    jnp.full((16,), -1, jnp.int32))
                    sl16 = pl.ds(g8 * 16, 16)
                    posb[sl16] = pos
                    dlocb[sl16] = d - lo
                    eidb[sl16] = jnp.full((16,), c * C, jnp.int32) + b16 + iota
                    return bv2 + t
                bv = lax.fori_loop(0, 8, g8_body, bv)
                pltpu.sync_copy(
                    dlocb, gd_l.at[plsc.Indices(posb, ignored_value=-1)])
                pltpu.sync_copy(
                    sbuf.at[pl.ds(sg * 128, 128)],
                    src_l.at[plsc.Indices(posb, ignored_value=-1)])
                pltpu.sync_copy(
                    eidb, eid_l.at[plsc.Indices(posb, ignored_value=-1)])
                return bv

            bvec = lax.fori_loop(0, C // 128, sg_body, bvec)

            exb[...] = bvec
            tv = exb[...]
            total_s = tv[0]
            nb = total_s // K

            def drain(i, _):
                process_batch(i, False)
                return 0
            lax.fori_loop(0, nb, drain, 0)

            @pl.when(nb > 0)
            def _():
                off_src = s_off + nb * K
                pltpu.sync_copy(gd_l.at[pl.ds(off_src, K)], gd_f)
                pltpu.sync_copy(gd_f, gd_l.at[pl.ds(s_off, K)])
                pltpu.sync_copy(src_l.at[pl.ds(off_src, K)], src_f)
                pltpu.sync_copy(src_f, src_l.at[pl.ds(s_off, K)])
                pltpu.sync_copy(eid_l.at[pl.ds(off_src, K)], eid_f)
                pltpu.sync_copy(eid_f, eid_l.at[pl.ds(s_off, K)])

            return bvec - jnp.full((16,), nb * K, jnp.int32)

        rem_vec = lax.fori_loop(0, NCHUNK, chunk_body, zl)

        process_batch(jnp.int32(0), True, rem_vec)

        pltpu.sync_copy(sumacc.at[pl.ds(0, RP)], sum_o.at[pl.ds(lo, RP)])
        pltpu.sync_copy(cntacc.at[pl.ds(0, RP)], cnt_o.at[pl.ds(lo, RP)])
        pltpu.sync_copy(mxacc.at[pl.ds(0, RP)], mx_o.at[pl.ds(lo, RP)])
        pltpu.sync_copy(mnacc.at[pl.ds(0, RP)], mn_o.at[pl.ds(lo, RP)])


def _sc_aggregate(edges, u, w):
    mesh = plsc.VectorSubcoreMesh(core_axis_name="c", subcore_axis_name="s")
    f32 = jnp.float32
    run = pl.kernel(
        _sc_body,
        out_type=(
            jax.ShapeDtypeStruct((N_OUT, D), f32),
            jax.ShapeDtypeStruct((N_OUT, D), f32),
            jax.ShapeDtypeStruct((N_OUT, D), f32),
            jax.ShapeDtypeStruct((N_CNT, 16), f32),
        ),
        mesh=mesh,
        scratch_types=[
            pltpu.VMEM((C,), jnp.int32),
            pltpu.VMEM((C,), jnp.int32),
            pltpu.VMEM((128,), jnp.int32),
            pltpu.VMEM((128,), jnp.int32),
            pltpu.VMEM((128,), jnp.int32),
            pltpu.VMEM_SHARED((NS * LCAP,), jnp.int32),
            pltpu.VMEM_SHARED((NS * LCAP,), jnp.int32),
            pltpu.VMEM_SHARED((NS * LCAP,), jnp.int32),
            pltpu.VMEM((K,), jnp.int32),
            pltpu.VMEM((K,), jnp.int32),
            pltpu.VMEM((K,), jnp.int32),
            pltpu.VMEM((K, D), f32),
            pltpu.VMEM((K, D), f32),
            pltpu.VMEM((RP + 8, D), f32),
            pltpu.VMEM((RP + 8, D), f32),
            pltpu.VMEM((RP + 8, D), f32),
            pltpu.VMEM((RP + 8, 16), f32),
            pltpu.VMEM((16,), jnp.int32),
            pltpu.SemaphoreType.DMA,
        ],
    )
    return run(edges, u, w)


def _uv_body(x_ref, w1_ref, w2_ref, u_ref, v_ref):
    x = x_ref[...]
    u_ref[...] = jnp.dot(x, w1_ref[...], preferred_element_type=jnp.float32)
    v_ref[...] = jnp.dot(x, w2_ref[...], preferred_element_type=jnp.float32)


def _uv_matmul(x, W1, W2):
    row = pl.BlockSpec((EPI_BLK, D), lambda i: (i, 0))
    full = pl.BlockSpec((D, D), lambda i: (0, 0))
    return pl.pallas_call(
        _uv_body,
        grid=(N // EPI_BLK,),
        in_specs=[row, full, full],
        out_specs=[row, row],
        out_shape=[jax.ShapeDtypeStruct((N, D), jnp.float32),
                   jax.ShapeDtypeStruct((N, D), jnp.float32)],
    )(x, W1, W2)


def _w_body(a_ref, w3_ref, o_ref):
    o_ref[...] = jnp.dot(a_ref[...], w3_ref[...],
                         preferred_element_type=jnp.float32)


def _w_matmul(edge_attr, W3):
    blk = 4000
    ed = edge_attr.shape[1]
    return pl.pallas_call(
        _w_body,
        grid=(E // blk,),
        in_specs=[pl.BlockSpec((blk, ed), lambda i: (i, 0)),
                  pl.BlockSpec((ed, D), lambda i: (0, 0))],
        out_specs=pl.BlockSpec((blk, D), lambda i: (i, 0)),
        out_shape=jax.ShapeDtypeStruct((E, D), jnp.float32),
    )(edge_attr, W3)


def _epilogue_body(x_ref, v_ref, sg_ref, mx_ref, mn_ref, cnt_ref,
                   wx_ref, wid_ref, wamp_ref, watt_ref, bpre_ref, bpost_ref,
                   out_ref):
    c = cnt_ref[...]
    has = c > 0.0
    cc = jnp.maximum(c, 1.0)
    vb = v_ref[...] + bpre_ref[...]
    mean = jnp.where(has, sg_ref[...] / cc + vb, 0.0)
    mx = jnp.where(has, mx_ref[...] + vb, 0.0)
    mn = jnp.where(has, mn_ref[...] + vb, 0.0)
    logd = jnp.log(c + 1.0)
    s1 = logd / AVG_D
    s2 = AVG_D / jnp.where(logd > 0.0, logd, 1.0)
    aggs = jnp.concatenate([mean, mx, mn], axis=1)
    x = x_ref[...]
    acc = x + bpost_ref[...]
    acc += jnp.dot(x, wx_ref[...], preferred_element_type=jnp.float32)
    acc += jnp.dot(aggs, wid_ref[...], preferred_element_type=jnp.float32)
    acc += s1 * jnp.dot(aggs, wamp_ref[...], preferred_element_type=jnp.float32)
    acc += s2 * jnp.dot(aggs, watt_ref[...], preferred_element_type=jnp.float32)
    out_ref[...] = acc


def _epilogue(x, v, sg, mx, mn, cnt, W_post, b_pre, b_post):
    Wx = W_post[0:D]
    Wid = W_post[D:4 * D]
    Wamp = W_post[4 * D:7 * D]
    Watt = W_post[7 * D:10 * D]
    row = pl.BlockSpec((EPI_BLK, D), lambda i: (i, 0))
    cnt_spec = pl.BlockSpec((EPI_BLK, 1), lambda i: (i, 0))
    full = lambda shape: pl.BlockSpec(shape, lambda i: (0, 0))
    return pl.pallas_call(
        _epilogue_body,
        grid=(N // EPI_BLK,),
        in_specs=[row, row, row, row, row, cnt_spec,
                  full((D, D)), full((3 * D, D)), full((3 * D, D)),
                  full((3 * D, D)), full((1, D)), full((1, D))],
        out_specs=row,
        out_shape=jax.ShapeDtypeStruct((N, D), jnp.float32),
    )(x, v, sg, mx, mn, cnt, Wx, Wid, Wamp, Watt,
      b_pre.reshape(1, D), b_post.reshape(1, D))


def kernel(x, edge_index, edge_attr, eig, W_pre, b_pre, W_post, b_post):
    src = edge_index[0]
    dst = edge_index[1]
    edges = jnp.concatenate(
        [dst, src, jnp.zeros((E_PAD - 2 * E,), jnp.int32)])
    W1 = W_pre[0:D]
    W2 = W_pre[D:2 * D]
    W3 = W_pre[2 * D:]
    u, v = _uv_matmul(x, W1, W2)
    w = _w_matmul(edge_attr, W3)
    u_pad = jnp.concatenate([u, jnp.zeros((N_OUT - N, D), jnp.float32)])
    sg, mx, mn, cnt = _sc_aggregate(edges, u_pad, w)
    return _epilogue(x, v, sg[:N], mx[:N], mn[:N], cnt[:N, 0].reshape(N, 1),
                     W_post, b_pre, b_post)


# speedup vs baseline: 1.0209x; 1.0209x over previous
import jax
import jax.numpy as jnp
from jax import lax
from jax.experimental import pallas as pl
from jax.experimental.pallas import tpu as pltpu
from jax.experimental.pallas import tpu_sc as plsc

N = 10000
E = 320000
D = 128
AVG_D = 3.5

NC = 2
NS = 16
NW = NC * NS
NPASS = 2
RP = 160
SPAN = NW * RP
N_PAD = NPASS * SPAN
C = 1280
NCHUNK = E // C
K = 128
LCAP = C + K
NEG = -3.4e38
POS = 3.4e38

N_OUT = 16512
N_CNT = 131200
E_PAD = 2200000

EPI_BLK = 1000


def _roll_gather(v, idx):
    return lax.gather(
        v, idx[:, None],
        dimension_numbers=lax.GatherDimensionNumbers(
            offset_dims=(), collapsed_slice_dims=(0,), start_index_map=(0,)),
        slice_sizes=(1,),
        mode=lax.GatherScatterMode.PROMISE_IN_BOUNDS)


def _sc_body(edges_h, u_h, w_h, sum_o, mx_o, mn_o, cnt_o,
             dbuf, sbuf, posb, dlocb, eidb, gd_l, src_l, eid_l,
             gd_f, src_f, eid_f, urows, wrows,
             sumacc, mxacc, mnacc, cntacc, exb, sem_g):
    c_ax = lax.axis_index("c")
    s_ax = lax.axis_index("s")
    wid = c_ax * NS + s_ax
    s_off = s_ax * LCAP
    iota = lax.iota(jnp.int32, 16)
    ones = jnp.full((16,), 1, jnp.int32)
    zl = jnp.full((16,), 0, jnp.int32)
    f32 = jnp.float32

    for p in range(NPASS):
        lo = p * SPAN + wid * RP
        hi = lo + RP

        def init_acc(r, _):
            for j in range(D // 16):
                sl = pl.ds(j * 16, 16)
                sumacc[r, sl] = jnp.full((16,), 0.0, f32)
                mxacc[r, sl] = jnp.full((16,), NEG, f32)
                mnacc[r, sl] = jnp.full((16,), POS, f32)
            cntacc[r, :] = jnp.full((16,), 0.0, f32)
            return 0
        lax.fori_loop(0, RP + 8, init_acc, 0)

        def process_batch(i, is_tail, rem_vec=None):
            off = s_off + i * K
            pltpu.sync_copy(gd_l.at[pl.ds(off, K)], gd_f)
            pltpu.sync_copy(src_l.at[pl.ds(off, K)], src_f)
            pltpu.sync_copy(eid_l.at[pl.ds(off, K)], eid_f)
            if is_tail:
                for j in range(K // 16):
                    sl = pl.ds(j * 16, 16)
                    lid = jnp.full((16,), j * 16, jnp.int32) + iota
                    valid = lid < rem_vec
                    gd_f[sl] = jnp.where(valid, gd_f[sl],
                                         jnp.full((16,), RP, jnp.int32))
                    src_f[sl] = jnp.where(valid, src_f[sl], zl)
                    eid_f[sl] = jnp.where(valid, eid_f[sl], zl)
            cp_u = pltpu.async_copy(u_h.at[src_f], urows, sem_g)
            cp_w = pltpu.async_copy(w_h.at[eid_f], wrows, sem_g)
            cp_u.wait()
            cp_w.wait()

            def q_body(q, _):
                gdv = gd_f[pl.ds(q * 16, 16)]
                ldv = jnp.minimum(jnp.maximum(gdv, 0), RP)
                exb[...] = ldv
                lvec = exb[...]
                for lane in range(16):
                    ld = lvec[lane]
                    row = q * 16 + lane
                    cntacc[ld, :] = cntacc[ld, :] + jnp.full((16,), 1.0, f32)
                    for j in range(D // 16):
                        sl = pl.ds(j * 16, 16)
                        gv = urows[row, sl] + wrows[row, sl]
                        sumacc[ld, sl] = sumacc[ld, sl] + gv
                        mxacc[ld, sl] = jnp.maximum(mxacc[ld, sl], gv)
                        mnacc[ld, sl] = jnp.minimum(mnacc[ld, sl], gv)
                return 0
            lax.fori_loop(0, K // 16, q_body, 0)

        def chunk_body(c, bvec):
            pltpu.sync_copy(edges_h.at[pl.ds(c * C, C)], dbuf)
            pltpu.sync_copy(edges_h.at[pl.ds(E + c * C, C)], sbuf)

            def sg_body(sg, bv):
                def g8_body(g8, bv2):
                    b16 = sg * 128 + g8 * 16
                    d = dbuf[pl.ds(b16, 16)]
                    m = (d >= lo) & (d < hi)
                    mi = jnp.where(m, ones, zl)
                    pr = mi
                    for k in (1, 2, 4, 8):
                        sh = _roll_gather(pr, jnp.maximum(iota - k, 0))
                        pr = pr + jnp.where(iota >= k, sh, zl)
                    t = _roll_gather(pr, jnp.full((16,), 15, jnp.int32))
                    pos = jnp.where(m, bv2 + pr - ones + s_off,
                                    jnp.full((16,), -1, jnp.int32))
                    sl16 = pl.ds(g8 * 16, 16)
                    posb[sl16] = pos
                    dlocb[sl16] = d - lo
                    eidb[sl16] = jnp.full((16,), c * C, jnp.int32) + b16 + iota
                    return bv2 + t
                bv = lax.fori_loop(0, 8, g8_body, bv)
                pltpu.sync_copy(
                    dlocb, gd_l.at[plsc.Indices(posb, ignored_value=-1)])
                pltpu.sync_copy(
                    sbuf.at[pl.ds(sg * 128, 128)],
                    src_l.at[plsc.Indices(posb, ignored_value=-1)])
                pltpu.sync_copy(
                    eidb, eid_l.at[plsc.Indices(posb, ignored_value=-1)])
                return bv

            bvec = lax.fori_loop(0, C // 128, sg_body, bvec)

            exb[...] = bvec
            tv = exb[...]
            total_s = tv[0]
            nb = total_s // K

            def drain(i, _):
                process_batch(i, False)
                return 0
            lax.fori_loop(0, nb, drain, 0)

            @pl.when(nb > 0)
            def _():
                off_src = s_off + nb * K
                pltpu.sync_copy(gd_l.at[pl.ds(off_src, K)], gd_f)
                pltpu.sync_copy(gd_f, gd_l.at[pl.ds(s_off, K)])
                pltpu.sync_copy(src_l.at[pl.ds(off_src, K)], src_f)
                pltpu.sync_copy(src_f, src_l.at[pl.ds(s_off, K)])
                pltpu.sync_copy(eid_l.at[pl.ds(off_src, K)], eid_f)
                pltpu.sync_copy(eid_f, eid_l.at[pl.ds(s_off, K)])

            return bvec - jnp.full((16,), nb * K, jnp.int32)

        rem_vec = lax.fori_loop(0, NCHUNK, chunk_body, zl)

        process_batch(jnp.int32(0), True, rem_vec)

        pltpu.sync_copy(sumacc.at[pl.ds(0, RP)], sum_o.at[pl.ds(lo, RP)])
        pltpu.sync_copy(cntacc.at[pl.ds(0, RP)], cnt_o.at[pl.ds(lo, RP)])
        pltpu.sync_copy(mxacc.at[pl.ds(0, RP)], mx_o.at[pl.ds(lo, RP)])
        pltpu.sync_copy(mnacc.at[pl.ds(0, RP)], mn_o.at[pl.ds(lo, RP)])


def _sc_aggregate(edges, u, w):
    mesh = plsc.VectorSubcoreMesh(core_axis_name="c", subcore_axis_name="s")
    f32 = jnp.float32
    run = pl.kernel(
        _sc_body,
        out_type=(
            jax.ShapeDtypeStruct((N_OUT, D), f32),
            jax.ShapeDtypeStruct((N_OUT, D), f32),
            jax.ShapeDtypeStruct((N_OUT, D), f32),
            jax.ShapeDtypeStruct((N_CNT, 16), f32),
        ),
        mesh=mesh,
        scratch_types=[
            pltpu.VMEM((C,), jnp.int32),
            pltpu.VMEM((C,), jnp.int32),
            pltpu.VMEM((128,), jnp.int32),
            pltpu.VMEM((128,), jnp.int32),
            pltpu.VMEM((128,), jnp.int32),
            pltpu.VMEM_SHARED((NS * LCAP,), jnp.int32),
            pltpu.VMEM_SHARED((NS * LCAP,), jnp.int32),
            pltpu.VMEM_SHARED((NS * LCAP,), jnp.int32),
            pltpu.VMEM((K,), jnp.int32),
            pltpu.VMEM((K,), jnp.int32),
            pltpu.VMEM((K,), jnp.int32),
            pltpu.VMEM((K, D), f32),
            pltpu.VMEM((K, D), f32),
            pltpu.VMEM((RP + 8, D), f32),
            pltpu.VMEM((RP + 8, D), f32),
            pltpu.VMEM((RP + 8, D), f32),
            pltpu.VMEM((RP + 8, 16), f32),
            pltpu.VMEM((16,), jnp.int32),
            pltpu.SemaphoreType.DMA,
        ],
    )
    return run(edges, u, w)


def _uv_body(x_ref, w1_ref, w2_ref, u_ref, v_ref):
    x = x_ref[...]
    u_ref[...] = jnp.dot(x, w1_ref[...], preferred_element_type=jnp.float32)
    v_ref[...] = jnp.dot(x, w2_ref[...], preferred_element_type=jnp.float32)


def _uv_matmul(x, W1, W2):
    row = pl.BlockSpec((EPI_BLK, D), lambda i: (i, 0))
    full = pl.BlockSpec((D, D), lambda i: (0, 0))
    return pl.pallas_call(
        _uv_body,
        grid=(N // EPI_BLK,),
        in_specs=[row, full, full],
        out_specs=[row, row],
        out_shape=[jax.ShapeDtypeStruct((N, D), jnp.float32),
                   jax.ShapeDtypeStruct((N, D), jnp.float32)],
    )(x, W1, W2)


def _w_body(a_ref, w3_ref, o_ref):
    o_ref[...] = jnp.dot(a_ref[...], w3_ref[...],
                         preferred_element_type=jnp.float32)


def _w_matmul(edge_attr, W3):
    blk = 4000
    ed = edge_attr.shape[1]
    return pl.pallas_call(
        _w_body,
        grid=(E // blk,),
        in_specs=[pl.BlockSpec((blk, ed), lambda i: (i, 0)),
                  pl.BlockSpec((ed, D), lambda i: (0, 0))],
        out_specs=pl.BlockSpec((blk, D), lambda i: (i, 0)),
        out_shape=jax.ShapeDtypeStruct((E, D), jnp.float32),
    )(edge_attr, W3)


def _epilogue_body(x_ref, v_ref, sg_ref, mx_ref, mn_ref, cnt_ref,
                   wx_ref, wid_ref, wamp_ref, watt_ref, bpre_ref, bpost_ref,
                   out_ref):
    c = cnt_ref[...]
    has = c > 0.0
    cc = jnp.maximum(c, 1.0)
    vb = v_ref[...] + bpre_ref[...]
    mean = jnp.where(has, sg_ref[...] / cc + vb, 0.0)
    mx = jnp.where(has, mx_ref[...] + vb, 0.0)
    mn = jnp.where(has, mn_ref[...] + vb, 0.0)
    logd = jnp.log(c + 1.0)
    s1 = logd / AVG_D
    s2 = AVG_D / jnp.where(logd > 0.0, logd, 1.0)
    aggs = jnp.concatenate([mean, mx, mn], axis=1)
    x = x_ref[...]
    acc = x + bpost_ref[...]
    acc += jnp.dot(x, wx_ref[...], preferred_element_type=jnp.float32)
    acc += jnp.dot(aggs, wid_ref[...], preferred_element_type=jnp.float32)
    acc += s1 * jnp.dot(aggs, wamp_ref[...], preferred_element_type=jnp.float32)
    acc += s2 * jnp.dot(aggs, watt_ref[...], preferred_element_type=jnp.float32)
    out_ref[...] = acc


def _epilogue(x, v, sg, mx, mn, cnt, W_post, b_pre, b_post):
    Wx = W_post[0:D]
    Wid = W_post[D:4 * D]
    Wamp = W_post[4 * D:7 * D]
    Watt = W_post[7 * D:10 * D]
    row = pl.BlockSpec((EPI_BLK, D), lambda i: (i, 0))
    cnt_spec = pl.BlockSpec((EPI_BLK, 1), lambda i: (i, 0))
    full = lambda shape: pl.BlockSpec(shape, lambda i: (0, 0))
    return pl.pallas_call(
        _epilogue_body,
        grid=(N // EPI_BLK,),
        in_specs=[row, row, row, row, row, cnt_spec,
                  full((D, D)), full((3 * D, D)), full((3 * D, D)),
                  full((3 * D, D)), full((1, D)), full((1, D))],
        out_specs=row,
        out_shape=jax.ShapeDtypeStruct((N, D), jnp.float32),
    )(x, v, sg, mx, mn, cnt, Wx, Wid, Wamp, Watt,
      b_pre.reshape(1, D), b_post.reshape(1, D))


def kernel(x, edge_index, edge_attr, eig, W_pre, b_pre, W_post, b_post):
    src = edge_index[0]
    dst = edge_index[1]
    edges = jnp.concatenate(
        [dst, src, jnp.zeros((E_PAD - 2 * E,), jnp.int32)])
    W1 = W_pre[0:D]
    W2 = W_pre[D:2 * D]
    W3 = W_pre[2 * D:]
    u, v = _uv_matmul(x, W1, W2)
    w = _w_matmul(edge_attr, W3)
    u_pad = jnp.concatenate([u, jnp.zeros((N_OUT - N, D), jnp.float32)])
    sg, mx, mn, cnt = _sc_aggregate(edges, u_pad, w)
    return _epilogue(x, v, sg[:N], mx[:N], mn[:N], cnt[:N, 0].reshape(N, 1),
                     W_post, b_pre, b_post)


# speedup vs baseline: 1.1499x; 1.1264x over previous
import jax
import jax.numpy as jnp
from jax import lax
from jax.experimental import pallas as pl
from jax.experimental.pallas import tpu as pltpu
from jax.experimental.pallas import tpu_sc as plsc

N = 10000
E = 320000
D = 128
AVG_D = 3.5

NC = 2
NS = 16
NW = NC * NS
NPASS = 2
RP = 160
SPAN = NW * RP
N_PAD = NPASS * SPAN
C = 1280
NCHUNK = E // C
K = 128
LCAP = C + K
NEG = -3.4e38
POS = 3.4e38

N_OUT = 16512
N_CNT = 131200
E_PAD = 2200000

EPI_BLK = 1000


def _roll_gather(v, idx):
    return lax.gather(
        v, idx[:, None],
        dimension_numbers=lax.GatherDimensionNumbers(
            offset_dims=(), collapsed_slice_dims=(0,), start_index_map=(0,)),
        slice_sizes=(1,),
        mode=lax.GatherScatterMode.PROMISE_IN_BOUNDS)


def _sc_body(edges_h, u_h, w_h, sum_o, mx_o, mn_o, cnt_o,
             dbuf, sbuf, posb, pkb, pk_l, src_l,
             pk_f, gd_f, src_f, eid_f, urows, wrows,
             sumacc, mxacc, mnacc, cntacc, exb, sem_g):
    c_ax = lax.axis_index("c")
    s_ax = lax.axis_index("s")
    wid = c_ax * NS + s_ax
    s_off = s_ax * LCAP
    iota = lax.iota(jnp.int32, 16)
    ones = jnp.full((16,), 1, jnp.int32)
    zl = jnp.full((16,), 0, jnp.int32)
    f32 = jnp.float32

    for p in range(NPASS):
        lo = p * SPAN + wid * RP
        hi = lo + RP

        def init_acc(r, _):
            for j in range(D // 16):
                sl = pl.ds(j * 16, 16)
                sumacc[r, sl] = jnp.full((16,), 0.0, f32)
                mxacc[r, sl] = jnp.full((16,), NEG, f32)
                mnacc[r, sl] = jnp.full((16,), POS, f32)
            cntacc[r, :] = jnp.full((16,), 0.0, f32)
            return 0
        lax.fori_loop(0, RP + 8, init_acc, 0)

        def process_batch(i, is_tail, rem_vec=None):
            off = s_off + i * K
            pltpu.sync_copy(pk_l.at[pl.ds(off, K)], pk_f)
            pltpu.sync_copy(src_l.at[pl.ds(off, K)], src_f)
            for j in range(K // 16):
                sl = pl.ds(j * 16, 16)
                pk = pk_f[sl]
                gd_f[sl] = lax.shift_right_logical(pk, 19)
                eid_f[sl] = pk & jnp.full((16,), (1 << 19) - 1, jnp.int32)
            if is_tail:
                for j in range(K // 16):
                    sl = pl.ds(j * 16, 16)
                    lid = jnp.full((16,), j * 16, jnp.int32) + iota
                    valid = lid < rem_vec
                    gd_f[sl] = jnp.where(valid, gd_f[sl],
                                         jnp.full((16,), RP, jnp.int32))
                    src_f[sl] = jnp.where(valid, src_f[sl], zl)
                    eid_f[sl] = jnp.where(valid, eid_f[sl], zl)
            cp_u = pltpu.async_copy(u_h.at[src_f], urows, sem_g)
            cp_w = pltpu.async_copy(w_h.at[eid_f], wrows, sem_g)
            cp_u.wait()
            cp_w.wait()

            def q_body(q, _):
                gdv = gd_f[pl.ds(q * 16, 16)]
                ldv = jnp.minimum(jnp.maximum(gdv, 0), RP)
                exb[...] = ldv
                lvec = exb[...]
                for lane in range(16):
                    ld = lvec[lane]
                    row = q * 16 + lane
                    cntacc[ld, :] = cntacc[ld, :] + jnp.full((16,), 1.0, f32)
                    for j in range(D // 16):
                        sl = pl.ds(j * 16, 16)
                        gv = urows[row, sl] + wrows[row, sl]
                        sumacc[ld, sl] = sumacc[ld, sl] + gv
                        mxacc[ld, sl] = jnp.maximum(mxacc[ld, sl], gv)
                        mnacc[ld, sl] = jnp.minimum(mnacc[ld, sl], gv)
                return 0
            lax.fori_loop(0, K // 16, q_body, 0)

        def chunk_body(c, bvec):
            pltpu.sync_copy(edges_h.at[pl.ds(c * C, C)], dbuf)
            pltpu.sync_copy(edges_h.at[pl.ds(E + c * C, C)], sbuf)

            def sg_body(sg, bv):
                def g8_body(g8, bv2):
                    b16 = sg * 128 + g8 * 16
                    d = dbuf[pl.ds(b16, 16)]
                    m = (d >= lo) & (d < hi)
                    mi = jnp.where(m, ones, zl)
                    pr = mi
                    for k in (1, 2, 4, 8):
                        sh = _roll_gather(pr, jnp.maximum(iota - k, 0))
                        pr = pr + jnp.where(iota >= k, sh, zl)
                    t = _roll_gather(pr, jnp.full((16,), 15, jnp.int32))
                    pos = jnp.where(m, bv2 + pr - ones + s_off,
                                    jnp.full((16,), -1, jnp.int32))
                    sl16 = pl.ds(g8 * 16, 16)
                    posb[sl16] = pos
                    eid = jnp.full((16,), c * C, jnp.int32) + b16 + iota
                    pkb[sl16] = lax.shift_left(d - lo, 19) | eid
                    return bv2 + t
                bv = lax.fori_loop(0, 8, g8_body, bv)
                pltpu.sync_copy(
                    pkb, pk_l.at[plsc.Indices(posb, ignored_value=-1)])
                pltpu.sync_copy(
                    sbuf.at[pl.ds(sg * 128, 128)],
                    src_l.at[plsc.Indices(posb, ignored_value=-1)])
                return bv

            bvec = lax.fori_loop(0, C // 128, sg_body, bvec)

            exb[...] = bvec
            tv = exb[...]
            total_s = tv[0]
            nb = total_s // K

            def drain(i, _):
                process_batch(i, False)
                return 0
            lax.fori_loop(0, nb, drain, 0)

            @pl.when(nb > 0)
            def _():
                off_src = s_off + nb * K
                pltpu.sync_copy(pk_l.at[pl.ds(off_src, K)], pk_f)
                pltpu.sync_copy(pk_f, pk_l.at[pl.ds(s_off, K)])
                pltpu.sync_copy(src_l.at[pl.ds(off_src, K)], src_f)
                pltpu.sync_copy(src_f, src_l.at[pl.ds(s_off, K)])

            return bvec - jnp.full((16,), nb * K, jnp.int32)

        rem_vec = lax.fori_loop(0, NCHUNK, chunk_body, zl)

        process_batch(jnp.int32(0), True, rem_vec)

        pltpu.sync_copy(sumacc.at[pl.ds(0, RP)], sum_o.at[pl.ds(lo, RP)])
        pltpu.sync_copy(cntacc.at[pl.ds(0, RP)], cnt_o.at[pl.ds(lo, RP)])
        pltpu.sync_copy(mxacc.at[pl.ds(0, RP)], mx_o.at[pl.ds(lo, RP)])
        pltpu.sync_copy(mnacc.at[pl.ds(0, RP)], mn_o.at[pl.ds(lo, RP)])


def _sc_aggregate(edges, u, w):
    mesh = plsc.VectorSubcoreMesh(core_axis_name="c", subcore_axis_name="s")
    f32 = jnp.float32
    run = pl.kernel(
        _sc_body,
        out_type=(
            jax.ShapeDtypeStruct((N_OUT, D), f32),
            jax.ShapeDtypeStruct((N_OUT, D), f32),
            jax.ShapeDtypeStruct((N_OUT, D), f32),
            jax.ShapeDtypeStruct((N_CNT, 16), f32),
        ),
        mesh=mesh,
        scratch_types=[
            pltpu.VMEM((C,), jnp.int32),
            pltpu.VMEM((C,), jnp.int32),
            pltpu.VMEM((128,), jnp.int32),
            pltpu.VMEM((128,), jnp.int32),
            pltpu.VMEM_SHARED((NS * LCAP,), jnp.int32),
            pltpu.VMEM_SHARED((NS * LCAP,), jnp.int32),
            pltpu.VMEM((K,), jnp.int32),
            pltpu.VMEM((K,), jnp.int32),
            pltpu.VMEM((K,), jnp.int32),
            pltpu.VMEM((K,), jnp.int32),
            pltpu.VMEM((K, D), f32),
            pltpu.VMEM((K, D), f32),
            pltpu.VMEM((RP + 8, D), f32),
            pltpu.VMEM((RP + 8, D), f32),
            pltpu.VMEM((RP + 8, D), f32),
            pltpu.VMEM((RP + 8, 16), f32),
            pltpu.VMEM((16,), jnp.int32),
            pltpu.SemaphoreType.DMA,
        ],
    )
    return run(edges, u, w)


def _uv_body(x_ref, w1_ref, w2_ref, u_ref, v_ref):
    x = x_ref[...]
    u_ref[...] = jnp.dot(x, w1_ref[...], preferred_element_type=jnp.float32)
    v_ref[...] = jnp.dot(x, w2_ref[...], preferred_element_type=jnp.float32)


def _uv_matmul(x, W1, W2):
    row = pl.BlockSpec((EPI_BLK, D), lambda i: (i, 0))
    full = pl.BlockSpec((D, D), lambda i: (0, 0))
    return pl.pallas_call(
        _uv_body,
        grid=(N // EPI_BLK,),
        in_specs=[row, full, full],
        out_specs=[row, row],
        out_shape=[jax.ShapeDtypeStruct((N, D), jnp.float32),
                   jax.ShapeDtypeStruct((N, D), jnp.float32)],
    )(x, W1, W2)


def _w_body(a_ref, w3_ref, o_ref):
    o_ref[...] = jnp.dot(a_ref[...], w3_ref[...],
                         preferred_element_type=jnp.float32)


def _w_matmul(edge_attr, W3):
    blk = 4000
    ed = edge_attr.shape[1]
    return pl.pallas_call(
        _w_body,
        grid=(E // blk,),
        in_specs=[pl.BlockSpec((blk, ed), lambda i: (i, 0)),
                  pl.BlockSpec((ed, D), lambda i: (0, 0))],
        out_specs=pl.BlockSpec((blk, D), lambda i: (i, 0)),
        out_shape=jax.ShapeDtypeStruct((E, D), jnp.float32),
    )(edge_attr, W3)


def _epilogue_body(x_ref, v_ref, sg_ref, mx_ref, mn_ref, cnt_ref,
                   wx_ref, wid_ref, wamp_ref, watt_ref, bpre_ref, bpost_ref,
                   out_ref):
    c = cnt_ref[...]
    has = c > 0.0
    cc = jnp.maximum(c, 1.0)
    vb = v_ref[...] + bpre_ref[...]
    mean = jnp.where(has, sg_ref[...] / cc + vb, 0.0)
    mx = jnp.where(has, mx_ref[...] + vb, 0.0)
    mn = jnp.where(has, mn_ref[...] + vb, 0.0)
    logd = jnp.log(c + 1.0)
    s1 = logd / AVG_D
    s2 = AVG_D / jnp.where(logd > 0.0, logd, 1.0)
    aggs = jnp.concatenate([mean, mx, mn], axis=1)
    x = x_ref[...]
    acc = x + bpost_ref[...]
    acc += jnp.dot(x, wx_ref[...], preferred_element_type=jnp.float32)
    acc += jnp.dot(aggs, wid_ref[...], preferred_element_type=jnp.float32)
    acc += s1 * jnp.dot(aggs, wamp_ref[...], preferred_element_type=jnp.float32)
    acc += s2 * jnp.dot(aggs, watt_ref[...], preferred_element_type=jnp.float32)
    out_ref[...] = acc


def _epilogue(x, v, sg, mx, mn, cnt, W_post, b_pre, b_post):
    Wx = W_post[0:D]
    Wid = W_post[D:4 * D]
    Wamp = W_post[4 * D:7 * D]
    Watt = W_post[7 * D:10 * D]
    row = pl.BlockSpec((EPI_BLK, D), lambda i: (i, 0))
    cnt_spec = pl.BlockSpec((EPI_BLK, 1), lambda i: (i, 0))
    full = lambda shape: pl.BlockSpec(shape, lambda i: (0, 0))
    return pl.pallas_call(
        _epilogue_body,
        grid=(N // EPI_BLK,),
        in_specs=[row, row, row, row, row, cnt_spec,
                  full((D, D)), full((3 * D, D)), full((3 * D, D)),
                  full((3 * D, D)), full((1, D)), full((1, D))],
        out_specs=row,
        out_shape=jax.ShapeDtypeStruct((N, D), jnp.float32),
    )(x, v, sg, mx, mn, cnt, Wx, Wid, Wamp, Watt,
      b_pre.reshape(1, D), b_post.reshape(1, D))


def kernel(x, edge_index, edge_attr, eig, W_pre, b_pre, W_post, b_post):
    src = edge_index[0]
    dst = edge_index[1]
    edges = jnp.concatenate(
        [dst, src, jnp.zeros((E_PAD - 2 * E,), jnp.int32)])
    W1 = W_pre[0:D]
    W2 = W_pre[D:2 * D]
    W3 = W_pre[2 * D:]
    u, v = _uv_matmul(x, W1, W2)
    w = _w_matmul(edge_attr, W3)
    u_pad = jnp.concatenate([u, jnp.zeros((N_OUT - N, D), jnp.float32)])
    sg, mx, mn, cnt = _sc_aggregate(edges, u_pad, w)
    return _epilogue(x, v, sg[:N], mx[:N], mn[:N], cnt[:N, 0].reshape(N, 1),
                     W_post, b_pre, b_post)


# speedup vs baseline: 1.2603x; 1.0960x over previous
import jax
import jax.numpy as jnp
from jax import lax
from jax.experimental import pallas as pl
from jax.experimental.pallas import tpu as pltpu
from jax.experimental.pallas import tpu_sc as plsc

N = 10000
E = 320000
D = 128
AVG_D = 3.5

NC = 2
NS = 16
NW = NC * NS
NPASS = 2
RP = 160
SPAN = NW * RP
N_PAD = NPASS * SPAN
C = 1280
NCHUNK = E // C
K = 128
LCAP = C + K
NEG = -3.4e38
POS = 3.4e38

N_OUT = 16512
N_CNT = 131200
E_PAD = 2200000

EPI_BLK = 1000


def _roll_gather(v, idx):
    return lax.gather(
        v, idx[:, None],
        dimension_numbers=lax.GatherDimensionNumbers(
            offset_dims=(), collapsed_slice_dims=(0,), start_index_map=(0,)),
        slice_sizes=(1,),
        mode=lax.GatherScatterMode.PROMISE_IN_BOUNDS)


def _sc_body(edges_h, u_h, w_h, sum_o, mx_o, mn_o, cnt_o,
             dbuf, sbuf, posb, pkb, pk_l, src_l,
             pk_f, gd_f, src_f, eid_f, urows, wrows,
             sumacc, mxacc, mnacc, cntacc, exb, sem_g):
    c_ax = lax.axis_index("c")
    s_ax = lax.axis_index("s")
    wid = c_ax * NS + s_ax
    s_off = s_ax * LCAP
    iota = lax.iota(jnp.int32, 16)
    ones = jnp.full((16,), 1, jnp.int32)
    zl = jnp.full((16,), 0, jnp.int32)
    f32 = jnp.float32

    for p in range(NPASS):
        lo = p * SPAN + wid * RP
        hi = lo + RP

        def init_acc(r, _):
            for j in range(D // 16):
                sl = pl.ds(j * 16, 16)
                sumacc[r, sl] = jnp.full((16,), 0.0, f32)
                mxacc[r, sl] = jnp.full((16,), NEG, f32)
                mnacc[r, sl] = jnp.full((16,), POS, f32)
            cntacc[r, :] = jnp.full((16,), 0.0, f32)
            return 0
        lax.fori_loop(0, RP + 8, init_acc, 0)

        def process_batch(i, is_tail, rem_vec=None):
            off = s_off + i * K
            pltpu.sync_copy(pk_l.at[pl.ds(off, K)], pk_f)
            pltpu.sync_copy(src_l.at[pl.ds(off, K)], src_f)
            for j in range(K // 16):
                sl = pl.ds(j * 16, 16)
                pk = pk_f[sl]
                gd_f[sl] = lax.shift_right_logical(pk, 19)
                eid_f[sl] = pk & jnp.full((16,), (1 << 19) - 1, jnp.int32)
            if is_tail:
                for j in range(K // 16):
                    sl = pl.ds(j * 16, 16)
                    lid = jnp.full((16,), j * 16, jnp.int32) + iota
                    valid = lid < rem_vec
                    gd_f[sl] = jnp.where(valid, gd_f[sl],
                                         jnp.full((16,), RP, jnp.int32))
                    src_f[sl] = jnp.where(valid, src_f[sl], zl)
                    eid_f[sl] = jnp.where(valid, eid_f[sl], zl)
            cp_u = pltpu.async_copy(u_h.at[src_f], urows, sem_g)
            cp_w = pltpu.async_copy(w_h.at[eid_f], wrows, sem_g)
            cp_u.wait()
            cp_w.wait()

            def q_body(q, _):
                gdv = gd_f[pl.ds(q * 16, 16)]
                ldv = jnp.minimum(jnp.maximum(gdv, 0), RP)
                exb[...] = ldv
                lvec = exb[...]
                for lane in range(16):
                    ld = lvec[lane]
                    row = q * 16 + lane
                    cntacc[ld, :] = cntacc[ld, :] + jnp.full((16,), 1.0, f32)
                    for j in range(D // 16):
                        sl = pl.ds(j * 16, 16)
                        gv = urows[row, sl] + wrows[row, sl]
                        sumacc[ld, sl] = sumacc[ld, sl] + gv
                        mxacc[ld, sl] = jnp.maximum(mxacc[ld, sl], gv)
                        mnacc[ld, sl] = jnp.minimum(mnacc[ld, sl], gv)
                return 0
            lax.fori_loop(0, K // 16, q_body, 0)

        def chunk_body(c, bvec):
            pltpu.sync_copy(edges_h.at[pl.ds(c * C, C)], dbuf)
            pltpu.sync_copy(edges_h.at[pl.ds(E + c * C, C)], sbuf)

            def sg_body(sg, bv):
                def g8_body(g8, bv2):
                    b16 = sg * 256 + g8 * 16
                    d = dbuf[pl.ds(b16, 16)]
                    m = (d >= lo) & (d < hi)
                    mi = jnp.where(m, ones, zl)
                    pr = mi
                    for k in (1, 2, 4, 8):
                        sh = _roll_gather(pr, jnp.maximum(iota - k, 0))
                        pr = pr + jnp.where(iota >= k, sh, zl)
                    t = _roll_gather(pr, jnp.full((16,), 15, jnp.int32))
                    pos = jnp.where(m, bv2 + pr - ones + s_off,
                                    jnp.full((16,), -1, jnp.int32))
                    sl16 = pl.ds(g8 * 16, 16)
                    posb[sl16] = pos
                    eid = jnp.full((16,), c * C, jnp.int32) + b16 + iota
                    pkb[sl16] = lax.shift_left(d - lo, 19) | eid
                    return bv2 + t
                bv = lax.fori_loop(0, 16, g8_body, bv)
                pltpu.sync_copy(
                    pkb, pk_l.at[plsc.Indices(posb, ignored_value=-1)])
                pltpu.sync_copy(
                    sbuf.at[pl.ds(sg * 256, 256)],
                    src_l.at[plsc.Indices(posb, ignored_value=-1)])
                return bv

            bvec = lax.fori_loop(0, C // 256, sg_body, bvec)

            exb[...] = bvec
            tv = exb[...]
            total_s = tv[0]
            nb = total_s // K

            def drain(i, _):
                process_batch(i, False)
                return 0
            lax.fori_loop(0, nb, drain, 0)

            @pl.when(nb > 0)
            def _():
                off_src = s_off + nb * K
                pltpu.sync_copy(pk_l.at[pl.ds(off_src, K)], pk_f)
                pltpu.sync_copy(pk_f, pk_l.at[pl.ds(s_off, K)])
                pltpu.sync_copy(src_l.at[pl.ds(off_src, K)], src_f)
                pltpu.sync_copy(src_f, src_l.at[pl.ds(s_off, K)])

            return bvec - jnp.full((16,), nb * K, jnp.int32)

        rem_vec = lax.fori_loop(0, NCHUNK, chunk_body, zl)

        process_batch(jnp.int32(0), True, rem_vec)

        pltpu.sync_copy(sumacc.at[pl.ds(0, RP)], sum_o.at[pl.ds(lo, RP)])
        pltpu.sync_copy(cntacc.at[pl.ds(0, RP)], cnt_o.at[pl.ds(lo, RP)])
        pltpu.sync_copy(mxacc.at[pl.ds(0, RP)], mx_o.at[pl.ds(lo, RP)])
        pltpu.sync_copy(mnacc.at[pl.ds(0, RP)], mn_o.at[pl.ds(lo, RP)])


def _sc_aggregate(edges, u, w):
    mesh = plsc.VectorSubcoreMesh(core_axis_name="c", subcore_axis_name="s")
    f32 = jnp.float32
    run = pl.kernel(
        _sc_body,
        out_type=(
            jax.ShapeDtypeStruct((N_OUT, D), f32),
            jax.ShapeDtypeStruct((N_OUT, D), f32),
            jax.ShapeDtypeStruct((N_OUT, D), f32),
            jax.ShapeDtypeStruct((N_CNT, 16), f32),
        ),
        mesh=mesh,
        scratch_types=[
            pltpu.VMEM((C,), jnp.int32),
            pltpu.VMEM((C,), jnp.int32),
            pltpu.VMEM((256,), jnp.int32),
            pltpu.VMEM((256,), jnp.int32),
            pltpu.VMEM_SHARED((NS * LCAP,), jnp.int32),
            pltpu.VMEM_SHARED((NS * LCAP,), jnp.int32),
            pltpu.VMEM((K,), jnp.int32),
            pltpu.VMEM((K,), jnp.int32),
            pltpu.VMEM((K,), jnp.int32),
            pltpu.VMEM((K,), jnp.int32),
            pltpu.VMEM((K, D), f32),
            pltpu.VMEM((K, D), f32),
            pltpu.VMEM((RP + 8, D), f32),
            pltpu.VMEM((RP + 8, D), f32),
            pltpu.VMEM((RP + 8, D), f32),
            pltpu.VMEM((RP + 8, 16), f32),
            pltpu.VMEM((16,), jnp.int32),
            pltpu.SemaphoreType.DMA,
        ],
    )
    return run(edges, u, w)


def _uv_body(x_ref, w1_ref, w2_ref, u_ref, v_ref):
    x = x_ref[...]
    u_ref[...] = jnp.dot(x, w1_ref[...], preferred_element_type=jnp.float32)
    v_ref[...] = jnp.dot(x, w2_ref[...], preferred_element_type=jnp.float32)


def _uv_matmul(x, W1, W2):
    row = pl.BlockSpec((EPI_BLK, D), lambda i: (i, 0))
    full = pl.BlockSpec((D, D), lambda i: (0, 0))
    return pl.pallas_call(
        _uv_body,
        grid=(N // EPI_BLK,),
        in_specs=[row, full, full],
        out_specs=[row, row],
        out_shape=[jax.ShapeDtypeStruct((N, D), jnp.float32),
                   jax.ShapeDtypeStruct((N, D), jnp.float32)],
    )(x, W1, W2)


def _w_body(a_ref, w3_ref, o_ref):
    o_ref[...] = jnp.dot(a_ref[...], w3_ref[...],
                         preferred_element_type=jnp.float32)


def _w_matmul(edge_attr, W3):
    blk = 4000
    ed = edge_attr.shape[1]
    return pl.pallas_call(
        _w_body,
        grid=(E // blk,),
        in_specs=[pl.BlockSpec((blk, ed), lambda i: (i, 0)),
                  pl.BlockSpec((ed, D), lambda i: (0, 0))],
        out_specs=pl.BlockSpec((blk, D), lambda i: (i, 0)),
        out_shape=jax.ShapeDtypeStruct((E, D), jnp.float32),
    )(edge_attr, W3)


def _epilogue_body(x_ref, v_ref, sg_ref, mx_ref, mn_ref, cnt_ref,
                   wx_ref, wid_ref, wamp_ref, watt_ref, bpre_ref, bpost_ref,
                   out_ref):
    c = cnt_ref[...]
    has = c > 0.0
    cc = jnp.maximum(c, 1.0)
    vb = v_ref[...] + bpre_ref[...]
    mean = jnp.where(has, sg_ref[...] / cc + vb, 0.0)
    mx = jnp.where(has, mx_ref[...] + vb, 0.0)
    mn = jnp.where(has, mn_ref[...] + vb, 0.0)
    logd = jnp.log(c + 1.0)
    s1 = logd / AVG_D
    s2 = AVG_D / jnp.where(logd > 0.0, logd, 1.0)
    aggs = jnp.concatenate([mean, mx, mn], axis=1)
    x = x_ref[...]
    acc = x + bpost_ref[...]
    acc += jnp.dot(x, wx_ref[...], preferred_element_type=jnp.float32)
    acc += jnp.dot(aggs, wid_ref[...], preferred_element_type=jnp.float32)
    acc += s1 * jnp.dot(aggs, wamp_ref[...], preferred_element_type=jnp.float32)
    acc += s2 * jnp.dot(aggs, watt_ref[...], preferred_element_type=jnp.float32)
    out_ref[...] = acc


def _epilogue(x, v, sg, mx, mn, cnt, W_post, b_pre, b_post):
    Wx = W_post[0:D]
    Wid = W_post[D:4 * D]
    Wamp = W_post[4 * D:7 * D]
    Watt = W_post[7 * D:10 * D]
    row = pl.BlockSpec((EPI_BLK, D), lambda i: (i, 0))
    cnt_spec = pl.BlockSpec((EPI_BLK, 1), lambda i: (i, 0))
    full = lambda shape: pl.BlockSpec(shape, lambda i: (0, 0))
    return pl.pallas_call(
        _epilogue_body,
        grid=(N // EPI_BLK,),
        in_specs=[row, row, row, row, row, cnt_spec,
                  full((D, D)), full((3 * D, D)), full((3 * D, D)),
                  full((3 * D, D)), full((1, D)), full((1, D))],
        out_specs=row,
        out_shape=jax.ShapeDtypeStruct((N, D), jnp.float32),
    )(x, v, sg, mx, mn, cnt, Wx, Wid, Wamp, Watt,
      b_pre.reshape(1, D), b_post.reshape(1, D))


def kernel(x, edge_index, edge_attr, eig, W_pre, b_pre, W_post, b_post):
    src = edge_index[0]
    dst = edge_index[1]
    edges = jnp.concatenate(
        [dst, src, jnp.zeros((E_PAD - 2 * E,), jnp.int32)])
    W1 = W_pre[0:D]
    W2 = W_pre[D:2 * D]
    W3 = W_pre[2 * D:]
    u, v = _uv_matmul(x, W1, W2)
    w = _w_matmul(edge_attr, W3)
    u_pad = jnp.concatenate([u, jnp.zeros((N_OUT - N, D), jnp.float32)])
    sg, mx, mn, cnt = _sc_aggregate(edges, u_pad, w)
    return _epilogue(x, v, sg[:N], mx[:N], mn[:N], cnt[:N, 0].reshape(N, 1),
                     W_post, b_pre, b_post)
